# baseline (speedup 1.0000x reference)
import functools

import jax
import jax.numpy as jnp
from jax import lax
from jax.experimental import pallas as pl
from jax.experimental.pallas import tpu as pltpu
from jax.experimental.pallas import tpu_sc as plsc

N_EMBED = 1024
MAX_SEQ_LEN = 8192
BATCH = 4


def _make_sc_broadcast():
    info = plsc.get_sparse_core_info()
    num_cores, num_subcores = info.num_cores, info.num_subcores
    num_workers = num_cores * num_subcores
    rows_per_worker = MAX_SEQ_LEN // num_workers

    mesh = plsc.VectorSubcoreMesh(core_axis_name="c", subcore_axis_name="s")

    chunk_rows = 32
    num_chunks = rows_per_worker // chunk_rows
    nbuf = 2

    scratch = [pltpu.VMEM((chunk_rows, N_EMBED), jnp.float32) for _ in range(nbuf)]
    scratch += [pltpu.SemaphoreType.DMA for _ in range(2 * nbuf)]

    @functools.partial(
        pl.kernel,
        mesh=mesh,
        out_type=jax.ShapeDtypeStruct((BATCH, MAX_SEQ_LEN, N_EMBED), jnp.float32),
        scratch_types=scratch,
    )
    def broadcast_rows(table_hbm, out_hbm, *scratch_refs):
        bufs = scratch_refs[:nbuf]
        rsems = scratch_refs[nbuf : 2 * nbuf]
        wsems = scratch_refs[2 * nbuf :]
        wid = lax.axis_index("s") * num_cores + lax.axis_index("c")
        base = wid * rows_per_worker

        reads = [None] * num_chunks
        writes = [None] * num_chunks
        reads[0] = pltpu.async_copy(
            table_hbm.at[pl.ds(base, chunk_rows)], bufs[0], rsems[0]
        )
        for i in range(num_chunks):
            j = i % nbuf
            reads[i].wait()
            writes[i] = [
                pltpu.async_copy(
                    bufs[j],
                    out_hbm.at[b, pl.ds(base + i * chunk_rows, chunk_rows)],
                    wsems[j],
                )
                for b in range(BATCH)
            ]
            if i + 1 < num_chunks:
                jn = (i + 1) % nbuf
                if i + 1 >= nbuf:
                    for h in writes[i + 1 - nbuf]:
                        h.wait()
                reads[i + 1] = pltpu.async_copy(
                    table_hbm.at[pl.ds(base + (i + 1) * chunk_rows, chunk_rows)],
                    bufs[jn],
                    rsems[jn],
                )
        for i in range(max(0, num_chunks - nbuf), num_chunks):
            for h in writes[i]:
                h.wait()

    return broadcast_rows


_sc_broadcast = _make_sc_broadcast()


def kernel(input_ids, pos_table):
    del input_ids
    return _sc_broadcast(pos_table)
